# baseline (speedup 1.0000x reference)
import jax
import jax.numpy as jnp
from jax import lax
from jax.experimental import pallas as pl
from jax.experimental.pallas import tpu as pltpu
from jax.experimental.pallas import tpu_sc as plsc

N_U = 10000
N_C = 10000
D = 128
TD = 128
E = 160000
NW = 32
EPW = E // NW
NVR2 = (EPW + 31) // 32
EBUF = NVR2 * 32
SEGP = 10240
NSEG = 3 * SEGP
SLW = SEGP // 10
TAIL = N_U - 9 * SLW
NTBL = N_U + 8
BN = 32
NB_FULL = SLW // BN
NB_EDGE = (TAIL - 16) // BN
NEGINF = float("-inf")

_mesh = plsc.VectorSubcoreMesh(core_axis_name="c", subcore_axis_name="s")
_sc_params = pltpu.CompilerParams(needs_layout_passes=False,
                                  use_tc_tiling_on_sc=False)


def _wid():
    return lax.axis_index("s") * 2 + lax.axis_index("c")


def _scan_body(src, dst, cas, tms, part_t, part_p,
               src_v, dst_v, cas_v, t_v, tt0, tt1, tt2, tp0, tp1, tp2, sem):
    w = _wid()
    base = w * EPW
    cps = [
        pltpu.async_copy(src.at[pl.ds(base, EPW)], src_v.at[pl.ds(0, EPW)], sem),
        pltpu.async_copy(dst.at[pl.ds(base, EPW)], dst_v.at[pl.ds(0, EPW)], sem),
        pltpu.async_copy(cas.at[pl.ds(base, EPW)], cas_v.at[pl.ds(0, EPW)], sem),
        pltpu.async_copy(tms.at[pl.ds(base, EPW)], t_v.at[pl.ds(0, EPW)], sem),
    ]
    minf = jnp.full((16,), NEGINF, jnp.float32)
    mneg = jnp.full((16,), -1, jnp.int32)

    def init_body(j, c):
        s = pl.ds(j * 16, 16)
        tt0[s] = minf
        tt1[s] = minf
        tt2[s] = minf
        tp0[s] = mneg
        tp1[s] = mneg
        tp2[s] = mneg
        return c

    lax.fori_loop(0, SEGP // 16, init_body, 0)
    for c in cps:
        c.wait()

    lane = lax.iota(jnp.int32, 16)

    def edge_body(i, c):
        for u in range(2):
            off = i * 32 + u * 16
            inb = (off + lane) < EPW
            t16 = jnp.where(inb, t_v[pl.ds(off, 16)], NEGINF)
            pos = jnp.where(inb, base + off + lane, 0)
            for idv, tt, tp in ((src_v, tt0, tp0), (dst_v, tt1, tp1),
                                (cas_v, tt2, tp2)):
                ids = jnp.where(inb, idv[pl.ds(off, 16)], SEGP - 1)
                ts, iss = plsc.sort_key_val(t16, ids)
                _, lastm = plsc.scan_count(iss)
                cur = plsc.load_gather(tt, [iss])
                plsc.store_scatter(tt, [iss], jnp.maximum(ts, cur), mask=lastm)
                nm = plsc.load_gather(tt, [ids])
                elig = t16 >= nm
                _, lm2 = plsc.scan_count(ids, mask=elig)
                m2 = lm2 & elig
                curp = plsc.load_gather(tp, [ids])
                plsc.store_scatter(tp, [ids], jnp.maximum(pos, curp), mask=m2)
        return c

    lax.fori_loop(0, NVR2, edge_body, 0)
    for s, tt in enumerate((tt0, tt1, tt2)):
        pltpu.sync_copy(tt, part_t.at[w, pl.ds(s * SEGP, SEGP)])
    for s, tp in enumerate((tp0, tp1, tp2)):
        pltpu.sync_copy(tp, part_p.at[w, pl.ds(s * SEGP, SEGP)])


def _meta_kernel_body(part_t, part_p, src, dst, cas, tms, pub, lu,
                      dt_o, t_o, val_o, idx_all,
                      pt_b, pp_b, bp_v, bpc_v, es_v, ed_v, ec_v, et_v, ep_v,
                      ls_v, ld_v, dt_v, tv_v, vl_v, idx_v, sem):
    w = _wid()

    @pl.when(w < 30)
    def _():
        stream = w // 10
        nbase = (w % 10) * SLW
        sbase = stream * SEGP + nbase
        cps = [pltpu.async_copy(part_t.at[r, pl.ds(sbase, SLW)], pt_b.at[r],
                                sem) for r in range(NW)]
        cps += [pltpu.async_copy(part_p.at[r, pl.ds(sbase, SLW)], pp_b.at[r],
                                 sem) for r in range(NW)]
        for c in cps:
            c.wait()

        def merge_body(j, c):
            o = pl.ds(j * 16, 16)
            bt = pt_b[0, o]
            for r in range(1, NW):
                bt = jnp.maximum(bt, pt_b[r, o])
            bp = jnp.full((16,), -1, jnp.int32)
            for r in range(NW):
                bp = jnp.maximum(bp, jnp.where(pt_b[r, o] >= bt, pp_b[r, o],
                                               -1))
            bp_v[o] = bp
            bpc_v[o] = jnp.maximum(bp, 0)
            return c

        lax.fori_loop(0, SLW // 16, merge_body, 0)

        CH = 128
        cps = []
        for k in range(SLW // CH):
            s = pl.ds(k * CH, CH)
            idx = bpc_v.at[s]
            cps += [
                pltpu.async_copy(src.at[idx], es_v.at[s], sem),
                pltpu.async_copy(dst.at[idx], ed_v.at[s], sem),
                pltpu.async_copy(cas.at[idx], ec_v.at[s], sem),
                pltpu.async_copy(tms.at[idx], et_v.at[s], sem),
                pltpu.async_copy(pub.at[idx], ep_v.at[s], sem),
            ]
        for c in cps:
            c.wait()
        cps = []
        for k in range(SLW // CH):
            s = pl.ds(k * CH, CH)
            cps += [
                pltpu.async_copy(lu.at[es_v.at[s]], ls_v.at[s], sem),
                pltpu.async_copy(lu.at[ed_v.at[s]], ld_v.at[s], sem),
            ]
        for c in cps:
            c.wait()

        s0m = jnp.broadcast_to(stream == 0, (16,))
        s1m = jnp.broadcast_to(stream == 1, (16,))

        def meta_body(j, c):
            o = pl.ds(j * 16, 16)
            bp16 = bp_v[o]
            valid = bp16 >= 0
            et16 = et_v[o]
            ref_t = jnp.where(s0m, ls_v[o], jnp.where(s1m, ld_v[o], ep_v[o]))
            dt_v[o] = jnp.where(valid, et16 - ref_t, 0.0)
            tv_v[o] = jnp.where(valid, et16, 0.0)
            vl_v[o] = jnp.where(valid, 1.0, 0.0)
            bi = 96 * (j // 2) + 16 * (j % 2)
            idx_v[pl.ds(bi, 16)] = jnp.where(valid, es_v[o], N_U)
            idx_v[pl.ds(bi + 32, 16)] = jnp.where(valid, ed_v[o], N_U)
            idx_v[pl.ds(bi + 64, 16)] = jnp.where(valid, ec_v[o] + NTBL,
                                                  N_U + NTBL)
            return c

        lax.fori_loop(0, SLW // 16, meta_body, 0)
        pltpu.sync_copy(dt_v, dt_o.at[stream, pl.ds(nbase, SLW)])
        pltpu.sync_copy(tv_v, t_o.at[stream, pl.ds(nbase, SLW)])
        pltpu.sync_copy(vl_v, val_o.at[stream, pl.ds(nbase, SLW)])
        pltpu.sync_copy(idx_v, idx_all.at[w])


def _gather_kernel_body(tbl, idx_all, raw0, raw1, raw2,
                        idx_v, g_a, g_b, gs_a, gs_b, ws_a, ws_b):
    w = _wid()

    @pl.when(w < 30)
    def _():
        stream = w // 10
        nbase = (w % 10) * SLW
        is_edge = (w % 10) == 9
        pltpu.sync_copy(idx_all.at[w], idx_v)
        raws = (raw0, raw1, raw2)

        def fire_gather(b, buf, gs):
            pltpu.async_copy(tbl.at[idx_v.at[pl.ds(b * 96, 96)]], buf, gs)

        def drain_gather(buf, gs):
            pltpu.make_async_copy(tbl.at[pl.ds(0, 96)], buf, gs).wait()

        def fire_writes(b, buf, ws):
            node = nbase + b * BN
            for c in range(3):
                pltpu.async_copy(buf.at[pl.ds(32 * c, 32)],
                                 raws[c].at[stream, pl.ds(node, 32)], ws)

        def drain_writes(buf, ws):
            for c in range(3):
                pltpu.make_async_copy(buf.at[pl.ds(32 * c, 32)],
                                      raws[c].at[stream, pl.ds(nbase, 32)],
                                      ws).wait()

        nb = jnp.where(is_edge, NB_EDGE, NB_FULL)
        fire_gather(0, g_a, gs_a)

        def pair_body(q, c):
            b0 = 2 * q

            @pl.when(q >= 1)
            def _():
                drain_writes(g_b, ws_b)

            fire_gather(b0 + 1, g_b, gs_b)
            drain_gather(g_a, gs_a)
            fire_writes(b0, g_a, ws_a)
            drain_writes(g_a, ws_a)

            @pl.when(b0 + 2 < nb)
            def _():
                fire_gather(b0 + 2, g_a, gs_a)

            drain_gather(g_b, gs_b)
            fire_writes(b0 + 1, g_b, ws_b)
            return c

        lax.fori_loop(0, nb // 2, pair_body, 0)
        drain_writes(g_b, ws_b)

        @pl.when(is_edge)
        def _():
            b = NB_EDGE
            node = nbase + b * BN
            pltpu.async_copy(tbl.at[idx_v.at[pl.ds(b * 96, 96)]], g_a,
                             gs_a).wait()
            for c in range(3):
                pltpu.sync_copy(g_a.at[pl.ds(32 * c, 16)],
                                raws[c].at[stream, pl.ds(node, 16)])


def _te_body(dtb, tb, vb, wref, bref, te_t, tc_t):
    s = pl.program_id(0)
    off = pl.multiple_of(s * SEGP, 128)
    dt = dtb[pl.ds(off, N_U)]
    v = vb[pl.ds(off, N_U)]
    wv = wref[s]
    bv = bref[s]
    te_t[0] = jnp.cos(wv[:, None] * dt[None, :] + bv[:, None]) * v[None, :]
    tc_t[0, 0] = tb[pl.ds(off, N_U)] * v


def kernel(source_nodes, destination_nodes, trans_cascades, edge_times,
           pub_times, user_state, cas_state, last_update, w_user, b_user,
           w_cas, b_cas):
    f32 = jnp.float32
    i32 = jnp.int32
    src = source_nodes.astype(i32)
    dst = destination_nodes.astype(i32)
    cas = trans_cascades.astype(i32)

    zrow = jnp.zeros((8, D), f32)
    tbl = jnp.concatenate([user_state, zrow, cas_state, zrow], axis=0)

    k1 = pl.kernel(
        _scan_body,
        out_type=(
            jax.ShapeDtypeStruct((NW, NSEG), f32),
            jax.ShapeDtypeStruct((NW, NSEG), i32),
        ),
        mesh=_mesh,
        compiler_params=_sc_params,
        scratch_types=[
            pltpu.VMEM((EBUF,), i32), pltpu.VMEM((EBUF,), i32),
            pltpu.VMEM((EBUF,), i32), pltpu.VMEM((EBUF,), f32),
            pltpu.VMEM((SEGP,), f32), pltpu.VMEM((SEGP,), f32),
            pltpu.VMEM((SEGP,), f32),
            pltpu.VMEM((SEGP,), i32), pltpu.VMEM((SEGP,), i32),
            pltpu.VMEM((SEGP,), i32),
            pltpu.SemaphoreType.DMA,
        ],
    )
    part_t, part_p = k1(src, dst, cas, edge_times)

    k2 = pl.kernel(
        _meta_kernel_body,
        out_type=(
            jax.ShapeDtypeStruct((3, SEGP), f32),
            jax.ShapeDtypeStruct((3, SEGP), f32),
            jax.ShapeDtypeStruct((3, SEGP), f32),
            jax.ShapeDtypeStruct((NW, 3 * SLW), i32),
        ),
        mesh=_mesh,
        compiler_params=_sc_params,
        scratch_types=[
            pltpu.VMEM((NW, SLW), f32), pltpu.VMEM((NW, SLW), i32),
            pltpu.VMEM((SLW,), i32), pltpu.VMEM((SLW,), i32),
            pltpu.VMEM((SLW,), i32), pltpu.VMEM((SLW,), i32),
            pltpu.VMEM((SLW,), i32), pltpu.VMEM((SLW,), f32),
            pltpu.VMEM((SLW,), f32), pltpu.VMEM((SLW,), f32),
            pltpu.VMEM((SLW,), f32), pltpu.VMEM((SLW,), f32),
            pltpu.VMEM((SLW,), f32), pltpu.VMEM((SLW,), f32),
            pltpu.VMEM((3 * SLW,), i32),
            pltpu.SemaphoreType.DMA,
        ],
    )
    dt_o, t_o, val_o, idx_all = k2(
        part_t, part_p, src, dst, cas, edge_times, pub_times, last_update)

    k3 = pl.kernel(
        _gather_kernel_body,
        out_type=(
            jax.ShapeDtypeStruct((3, N_U, D), f32),
            jax.ShapeDtypeStruct((3, N_U, D), f32),
            jax.ShapeDtypeStruct((3, N_U, D), f32),
        ),
        mesh=_mesh,
        compiler_params=_sc_params,
        scratch_types=[
            pltpu.VMEM((3 * SLW,), i32),
            pltpu.VMEM((96, D), f32), pltpu.VMEM((96, D), f32),
            pltpu.SemaphoreType.DMA, pltpu.SemaphoreType.DMA,
            pltpu.SemaphoreType.DMA, pltpu.SemaphoreType.DMA,
        ],
    )
    raw0, raw1, raw2 = k3(tbl, idx_all)

    wu2 = w_user.reshape(1, TD)
    bu2 = b_user.reshape(1, TD)
    wc2 = w_cas.reshape(1, TD)
    bc2 = b_cas.reshape(1, TD)
    wall = jnp.concatenate([wu2, wu2, wc2], axis=0)
    ball = jnp.concatenate([bu2, bu2, bc2], axis=0)

    te_t, tc_t = pl.pallas_call(
        _te_body,
        out_shape=(
            jax.ShapeDtypeStruct((3, TD, N_U), f32),
            jax.ShapeDtypeStruct((3, 1, N_U), f32),
        ),
        grid=(3,),
        in_specs=[
            pl.BlockSpec((3 * SEGP,), lambda s: (0,)),
            pl.BlockSpec((3 * SEGP,), lambda s: (0,)),
            pl.BlockSpec((3 * SEGP,), lambda s: (0,)),
            pl.BlockSpec((3, TD), lambda s: (0, 0)),
            pl.BlockSpec((3, TD), lambda s: (0, 0)),
        ],
        out_specs=[
            pl.BlockSpec((1, TD, N_U), lambda s: (s, 0, 0)),
            pl.BlockSpec((1, 1, N_U), lambda s: (s, 0, 0)),
        ],
    )(dt_o.reshape(3 * SEGP), t_o.reshape(3 * SEGP), val_o.reshape(3 * SEGP),
      wall, ball)

    te = jnp.transpose(te_t, (0, 2, 1))
    tcol = jnp.transpose(tc_t, (0, 2, 1))
    return jnp.concatenate([raw0, raw1, raw2, te, tcol], axis=-1)
